# baseline (speedup 1.0000x reference)
import functools

import jax
import jax.numpy as jnp
from jax import lax
from jax.experimental import pallas as pl
from jax.experimental.pallas import tpu as pltpu
from jax.experimental.pallas import tpu_sc as plsc

_VOCAB = 1000000
_D = 64
_B = 4096
_H = 200
_TOT = _B * _H
_NW = 32
_PER_W = _TOT // _NW
_C = 512
_NCH = _PER_W // _C


def _make_kernel():
    mesh = plsc.VectorSubcoreMesh(core_axis_name="c", subcore_axis_name="s")

    @functools.partial(
        pl.kernel,
        mesh=mesh,
        out_type=jax.ShapeDtypeStruct((_TOT, _D), jnp.float32),
        scratch_types=[
            pltpu.VMEM((_C,), jnp.int32),
            pltpu.VMEM((_C, _D), jnp.float32),
            pltpu.SemaphoreType.DMA,
            pltpu.SemaphoreType.DMA,
        ],
        compiler_params=pltpu.CompilerParams(use_tc_tiling_on_sc=False),
    )
    def emb(x_hbm, table_hbm, out_hbm, idx_v, rows, gsem, ssem):
        wid = lax.axis_index("s") * 2 + lax.axis_index("c")
        base = wid * _PER_W

        def body(j, carry):
            pltpu.sync_copy(x_hbm.at[wid * _NCH + j], idx_v)
            pltpu.async_copy(table_hbm.at[idx_v], rows, gsem).wait()
            pltpu.async_copy(
                rows, out_hbm.at[pl.ds(base + j * _C, _C)], ssem
            ).wait()
            return carry

        lax.fori_loop(0, _NCH, body, 0)

    return emb


_emb = _make_kernel()


def kernel(x, table):
    xf = x.reshape(_NW * _NCH, _C).astype(jnp.int32)
    out = _emb(xf, table)
    return out.reshape(_B, _H, _D)


# speedup vs baseline: 1.0388x; 1.0388x over previous
import functools

import jax
import jax.numpy as jnp
from jax import lax
from jax.experimental import pallas as pl
from jax.experimental.pallas import tpu as pltpu
from jax.experimental.pallas import tpu_sc as plsc

_VOCAB = 1000000
_D = 64
_B = 4096
_H = 200
_TOT = _B * _H
_NW = 32
_PER_W = _TOT // _NW
_C = 640
_NCH = _PER_W // _C


def _make_kernel():
    mesh = plsc.VectorSubcoreMesh(core_axis_name="c", subcore_axis_name="s")

    @functools.partial(
        pl.kernel,
        mesh=mesh,
        out_type=jax.ShapeDtypeStruct((_TOT, _D), jnp.float32),
        scratch_types=[
            pltpu.VMEM((_PER_W,), jnp.int32),
            pltpu.VMEM((_C, _D), jnp.float32),
            pltpu.VMEM((_C, _D), jnp.float32),
            pltpu.SemaphoreType.DMA,
            pltpu.SemaphoreType.DMA,
            pltpu.SemaphoreType.DMA,
            pltpu.SemaphoreType.DMA,
        ],
        compiler_params=pltpu.CompilerParams(use_tc_tiling_on_sc=False),
    )
    def emb(x_hbm, table_hbm, out_hbm, idx_all, rows0, rows1, g0, g1, s0, s1):
        wid = lax.axis_index("s") * 2 + lax.axis_index("c")
        base = wid * _PER_W
        pltpu.sync_copy(x_hbm.at[wid], idx_all)

        rows = (rows0, rows1)
        gsem = (g0, g1)
        ssem = (s0, s1)

        def idx_slice(j):
            return idx_all.at[pl.ds(pl.multiple_of(j * _C, _C), _C)]

        def start_gather(j, b):
            pltpu.async_copy(table_hbm.at[idx_slice(j)], rows[b], gsem[b])

        def start_store(j, b):
            pltpu.async_copy(
                rows[b], out_hbm.at[pl.ds(base + j * _C, _C)], ssem[b]
            )

        def wait_gather(b):
            pltpu.make_async_copy(
                table_hbm.at[idx_slice(0)], rows[b], gsem[b]
            ).wait()

        def wait_store(b):
            pltpu.make_async_copy(
                rows[b], out_hbm.at[pl.ds(base, _C)], ssem[b]
            ).wait()

        start_gather(0, 0)

        def body(i, carry):
            for b in (0, 1):
                j = 2 * i + b
                nb = 1 - b
                wait_gather(b)

                @pl.when(j >= 1)
                def _():
                    wait_store(nb)

                @pl.when(j + 1 < _NCH)
                def _():
                    start_gather(j + 1, nb)

                start_store(j, b)
            return carry

        lax.fori_loop(0, _NCH // 2, body, 0)
        wait_store((_NCH - 1) % 2)

    return emb


_emb = _make_kernel()


def kernel(x, table):
    xf = x.reshape(_NW, _PER_W).astype(jnp.int32)
    out = _emb(xf, table)
    return out.reshape(_B, _H, _D)
